# baseline (speedup 1.0000x reference)
import functools

import numpy as np
import jax
import jax.numpy as jnp
from jax import lax
from jax.experimental import pallas as pl
from jax.experimental.pallas import tpu as pltpu
from jax.experimental.pallas import tpu_sc as plsc

B, S, L = 64, 400, 16
VW, DW = 100000, 128
VC, DC = 96, 64
NF = 128
KW = 5
D = DW + NF
N = B * S
NW = 32
RPW = N // NW
TT = 512
NPOS = L - KW + 1


def _word_gather(idx_flat, table):
    mesh = plsc.VectorSubcoreMesh(core_axis_name="c", subcore_axis_name="s")

    @functools.partial(
        pl.kernel,
        out_type=jax.ShapeDtypeStruct((N, DW), jnp.float32),
        mesh=mesh,
        scratch_types=[
            pltpu.VMEM((RPW,), jnp.int32),
            pltpu.VMEM((RPW, DW), jnp.float32),
            pltpu.SemaphoreType.DMA,
        ],
    )
    def gk(idx_hbm, table_hbm, out_hbm, idx_v, rows_v, sem):
        wid = lax.axis_index("s") * 2 + lax.axis_index("c")
        base = wid * RPW
        pltpu.sync_copy(idx_hbm.at[pl.ds(base, RPW)], idx_v)
        pltpu.async_copy(table_hbm.at[idx_v], rows_v, sem).wait()
        pltpu.sync_copy(rows_v, out_hbm.at[pl.ds(base, RPW)])

    return gk(idx_flat, table)


def _char_body(chars_ref, ctab_ref, wcat_ref, bconv_ref, exp_ref, cmod_ref,
               out_ref, m_ref):
    @pl.when(pl.program_id(0) == 0)
    def _():
        m_ref[...] = jnp.zeros(((KW + 1) * NF, 2 * NF), jnp.bfloat16)
        mt = jnp.dot(ctab_ref[...], wcat_ref[...],
                     preferred_element_type=jnp.float32)
        for k in range(KW):
            blk = mt[:, k * NF:(k + 1) * NF].astype(jnp.bfloat16)
            m_ref[pl.ds(k * NF, VC), 0:NF] = blk
            m_ref[pl.ds((k + 1) * NF, VC), NF:2 * NF] = blk

    chars_bf = chars_ref[...].astype(jnp.bfloat16)
    chars_rep = jnp.dot(chars_bf, exp_ref[...],
                        preferred_element_type=jnp.float32
                        ).astype(jnp.bfloat16)
    oh2 = jnp.where(chars_rep == cmod_ref[...],
                    jnp.bfloat16(1), jnp.bfloat16(0))
    m = m_ref[...]

    acc = None
    for p in range(NPOS // 2):
        pr = lax.dot_general(oh2[:, 2 * p * NF:(2 * p + KW + 1) * NF], m,
                             (((1,), (0,)), ((), ())),
                             preferred_element_type=jnp.float32)
        acc = pr if acc is None else jnp.maximum(acc, pr)
    acc = jnp.maximum(acc[:, :NF], acc[:, NF:])
    out_ref[...] = jnp.maximum(acc + bconv_ref[...], 0.0)


def _char_emb(chars2, ctab, wcat, bconv, expander, cmod):
    const = lambda i: (0, 0)
    row = lambda i: (i, 0)
    return pl.pallas_call(
        _char_body,
        grid=(N // TT,),
        in_specs=[
            pl.BlockSpec((TT, L), row),
            pl.BlockSpec((VC, DC), const),
            pl.BlockSpec((DC, KW * NF), const),
            pl.BlockSpec((1, NF), const),
            pl.BlockSpec((L, L * NF), const),
            pl.BlockSpec((1, L * NF), const),
        ],
        out_specs=pl.BlockSpec((TT, NF), row),
        out_shape=jax.ShapeDtypeStruct((N, NF), jnp.float32),
        scratch_shapes=[pltpu.VMEM(((KW + 1) * NF, 2 * NF), jnp.bfloat16)],
    )(chars2, ctab, wcat, bconv, expander, cmod)


def _hw_body(words_ref, ce_ref,
             wg0_ref, bg0_ref, wt0_ref, bt0_ref,
             wg1_ref, bg1_ref, wt1_ref, bt1_ref,
             out_ref):
    x = jnp.concatenate([words_ref[...], ce_ref[...]], axis=1)
    for wg, bg, wt, bt in ((wg0_ref, bg0_ref, wt0_ref, bt0_ref),
                           (wg1_ref, bg1_ref, wt1_ref, bt1_ref)):
        xb = x.astype(jnp.bfloat16)
        g = jax.nn.sigmoid(
            jnp.dot(xb, wg[...], preferred_element_type=jnp.float32) + bg[...])
        t = jnp.maximum(
            jnp.dot(xb, wt[...], preferred_element_type=jnp.float32) + bt[...],
            0.0)
        x = g * t + (1.0 - g) * x
    out_ref[...] = x


def _highway(word_rows, ce, wg0t, bg0, wt0t, bt0, wg1t, bg1, wt1t, bt1):
    const = lambda i: (0, 0)
    row = lambda i: (i, 0)
    return pl.pallas_call(
        _hw_body,
        grid=(N // TT,),
        in_specs=[
            pl.BlockSpec((TT, DW), row),
            pl.BlockSpec((TT, NF), row),
            pl.BlockSpec((D, D), const),
            pl.BlockSpec((1, D), const),
            pl.BlockSpec((D, D), const),
            pl.BlockSpec((1, D), const),
            pl.BlockSpec((D, D), const),
            pl.BlockSpec((1, D), const),
            pl.BlockSpec((D, D), const),
            pl.BlockSpec((1, D), const),
        ],
        out_specs=pl.BlockSpec((TT, D), row),
        out_shape=jax.ShapeDtypeStruct((N, D), jnp.float32),
    )(word_rows, ce, wg0t, bg0, wt0t, bt0, wg1t, bg1, wt1t, bt1)


def kernel(word_idxs, char_idxs, word_table, char_table, W_conv, b_conv,
           Wt0, bt0, Wg0, bg0, Wt1, bt1, Wg1, bg1):
    widx = word_idxs.reshape(N).astype(jnp.int32)
    word_rows = _word_gather(widx, word_table)

    chars2 = char_idxs.reshape(N, L).astype(jnp.int32)
    wcat = jnp.transpose(W_conv[:, :, 0, :], (1, 2, 0)).reshape(DC, KW * NF)
    bconv = b_conv.reshape(1, NF)
    cols = np.arange(L * NF)
    expander = jnp.asarray(
        (cols // NF == np.arange(L)[:, None]).astype(np.float32),
        dtype=jnp.bfloat16)
    cmod = jnp.asarray((cols % NF).astype(np.float32)[None, :],
                       dtype=jnp.bfloat16)
    ce = _char_emb(chars2, char_table, wcat, bconv, expander, cmod)

    emb = _highway(
        word_rows, ce,
        Wg0.T.astype(jnp.bfloat16), bg0.reshape(1, D),
        Wt0.T.astype(jnp.bfloat16), bt0.reshape(1, D),
        Wg1.T.astype(jnp.bfloat16), bg1.reshape(1, D),
        Wt1.T.astype(jnp.bfloat16), bt1.reshape(1, D),
    )
    return emb.reshape(B, S, D)


# speedup vs baseline: 1.2026x; 1.2026x over previous
import functools

import numpy as np
import jax
import jax.numpy as jnp
from jax import lax
from jax.experimental import pallas as pl
from jax.experimental.pallas import tpu as pltpu
from jax.experimental.pallas import tpu_sc as plsc

B, S, L = 64, 400, 16
VW, DW = 100000, 128
VC, DC = 96, 64
NF = 128
KW = 5
D = DW + NF
N = B * S
NW = 32
RPW = N // NW
TT = 1024
NPOS = L - KW + 1


def _word_gather(idx_flat, table):
    mesh = plsc.VectorSubcoreMesh(core_axis_name="c", subcore_axis_name="s")

    @functools.partial(
        pl.kernel,
        out_type=jax.ShapeDtypeStruct((N, DW), jnp.float32),
        mesh=mesh,
        scratch_types=[
            pltpu.VMEM((RPW,), jnp.int32),
            pltpu.VMEM((RPW, DW), jnp.float32),
            pltpu.SemaphoreType.DMA,
        ],
    )
    def gk(idx_hbm, table_hbm, out_hbm, idx_v, rows_v, sem):
        wid = lax.axis_index("s") * 2 + lax.axis_index("c")
        base = wid * RPW
        pltpu.sync_copy(idx_hbm.at[pl.ds(base, RPW)], idx_v)
        pltpu.async_copy(table_hbm.at[idx_v], rows_v, sem).wait()
        pltpu.sync_copy(rows_v, out_hbm.at[pl.ds(base, RPW)])

    return gk(idx_flat, table)


def _dot_t(x, w):
    return lax.dot_general(x, w, (((1,), (1,)), ((), ())),
                           preferred_element_type=jnp.float32)


def _tc_body(chars_ref, words_ref, ctab_ref, wcat_ref, bconv_ref,
             exp_ref, cmod_ref,
             wg0_ref, bg0_ref, wt0_ref, bt0_ref,
             wg1_ref, bg1_ref, wt1_ref, bt1_ref,
             out_ref, m_ref):
    @pl.when(pl.program_id(0) == 0)
    def _():
        m_ref[...] = jnp.zeros(((KW + 1) * NF, 2 * NF), jnp.bfloat16)
        mt = jnp.dot(ctab_ref[...], wcat_ref[...],
                     preferred_element_type=jnp.float32)
        for k in range(KW):
            blk = mt[:, k * NF:(k + 1) * NF].astype(jnp.bfloat16)
            m_ref[pl.ds(k * NF, VC), 0:NF] = blk
            m_ref[pl.ds((k + 1) * NF, VC), NF:2 * NF] = blk

    chars_bf = chars_ref[...].astype(jnp.bfloat16)
    chars_rep = jnp.dot(chars_bf, exp_ref[...],
                        preferred_element_type=jnp.float32
                        ).astype(jnp.bfloat16)
    oh2 = jnp.where(chars_rep == cmod_ref[...],
                    jnp.bfloat16(1), jnp.bfloat16(0))
    m = m_ref[...]

    acc = None
    for p in range(NPOS // 2):
        pr = lax.dot_general(oh2[:, 2 * p * NF:(2 * p + KW + 1) * NF], m,
                             (((1,), (0,)), ((), ())),
                             preferred_element_type=jnp.float32)
        acc = pr if acc is None else jnp.maximum(acc, pr)
    acc = jnp.maximum(acc[:, :NF], acc[:, NF:])
    ce = jnp.maximum(acc + bconv_ref[...], 0.0)

    x = jnp.concatenate([words_ref[...], ce], axis=1)
    for wg, bg, wt, bt in ((wg0_ref, bg0_ref, wt0_ref, bt0_ref),
                           (wg1_ref, bg1_ref, wt1_ref, bt1_ref)):
        xb = x.astype(jnp.bfloat16)
        g = jax.nn.sigmoid(_dot_t(xb, wg[...]) + bg[...])
        t = jnp.maximum(_dot_t(xb, wt[...]) + bt[...], 0.0)
        x = g * t + (1.0 - g) * x
    out_ref[...] = x


def _tc_fused(chars2, word_rows, ctab, wcat, bconv, expander, cmod,
              wg0, bg0, wt0, bt0, wg1, bg1, wt1, bt1):
    const = lambda i: (0, 0)
    row = lambda i: (i, 0)
    return pl.pallas_call(
        _tc_body,
        grid=(N // TT,),
        in_specs=[
            pl.BlockSpec((TT, L), row),
            pl.BlockSpec((TT, DW), row),
            pl.BlockSpec((VC, DC), const),
            pl.BlockSpec((DC, KW * NF), const),
            pl.BlockSpec((1, NF), const),
            pl.BlockSpec((L, L * NF), const),
            pl.BlockSpec((1, L * NF), const),
            pl.BlockSpec((D, D), const),
            pl.BlockSpec((1, D), const),
            pl.BlockSpec((D, D), const),
            pl.BlockSpec((1, D), const),
            pl.BlockSpec((D, D), const),
            pl.BlockSpec((1, D), const),
            pl.BlockSpec((D, D), const),
            pl.BlockSpec((1, D), const),
        ],
        out_specs=pl.BlockSpec((TT, D), row),
        out_shape=jax.ShapeDtypeStruct((N, D), jnp.float32),
        scratch_shapes=[pltpu.VMEM(((KW + 1) * NF, 2 * NF), jnp.bfloat16)],
    )(chars2, word_rows, ctab, wcat, bconv, expander, cmod,
      wg0, bg0, wt0, bt0, wg1, bg1, wt1, bt1)


def kernel(word_idxs, char_idxs, word_table, char_table, W_conv, b_conv,
           Wt0, bt0, Wg0, bg0, Wt1, bt1, Wg1, bg1):
    widx = word_idxs.reshape(N).astype(jnp.int32)
    word_rows = _word_gather(widx, word_table)

    chars2 = char_idxs.reshape(N, L).astype(jnp.int32)
    wcat = jnp.transpose(W_conv[:, :, 0, :], (1, 2, 0)).reshape(DC, KW * NF)
    bconv = b_conv.reshape(1, NF)
    cols = np.arange(L * NF)
    expander = jnp.asarray(
        (cols // NF == np.arange(L)[:, None]).astype(np.float32),
        dtype=jnp.bfloat16)
    cmod = jnp.asarray((cols % NF).astype(np.float32)[None, :],
                       dtype=jnp.bfloat16)
    emb = _tc_fused(
        chars2, word_rows, char_table, wcat, bconv, expander, cmod,
        Wg0.astype(jnp.bfloat16), bg0.reshape(1, D),
        Wt0.astype(jnp.bfloat16), bt0.reshape(1, D),
        Wg1.astype(jnp.bfloat16), bg1.reshape(1, D),
        Wt1.astype(jnp.bfloat16), bt1.reshape(1, D),
    )
    return emb.reshape(B, S, D)


# speedup vs baseline: 1.2057x; 1.0025x over previous
import functools

import numpy as np
import jax
import jax.numpy as jnp
from jax import lax
from jax.experimental import pallas as pl
from jax.experimental.pallas import tpu as pltpu
from jax.experimental.pallas import tpu_sc as plsc

B, S, L = 64, 400, 16
VW, DW = 100000, 128
VC, DC = 96, 64
NF = 128
KW = 5
D = DW + NF
N = B * S
NW = 32
RPW = N // NW
TT = 1024
NPOS = L - KW + 1


def _word_gather(idx_flat, table):
    mesh = plsc.VectorSubcoreMesh(core_axis_name="c", subcore_axis_name="s")

    half = RPW // 2

    @functools.partial(
        pl.kernel,
        out_type=jax.ShapeDtypeStruct((N, DW), jnp.float32),
        mesh=mesh,
        scratch_types=[
            pltpu.VMEM((RPW,), jnp.int32),
            pltpu.VMEM((half, DW), jnp.float32),
            pltpu.VMEM((half, DW), jnp.float32),
            pltpu.SemaphoreType.DMA,
            pltpu.SemaphoreType.DMA,
            pltpu.SemaphoreType.DMA,
            pltpu.SemaphoreType.DMA,
        ],
    )
    def gk(idx_hbm, table_hbm, out_hbm, idx_v, ra, rb, sga, sgb, ssa, ssb):
        wid = lax.axis_index("s") * 2 + lax.axis_index("c")
        base = wid * RPW
        pltpu.sync_copy(idx_hbm.at[pl.ds(base, RPW)], idx_v)
        ga = pltpu.async_copy(table_hbm.at[idx_v.at[pl.ds(0, half)]], ra, sga)
        gb = pltpu.async_copy(table_hbm.at[idx_v.at[pl.ds(half, half)]], rb, sgb)
        ga.wait()
        sa = pltpu.async_copy(ra, out_hbm.at[pl.ds(base, half)], ssa)
        gb.wait()
        sb = pltpu.async_copy(rb, out_hbm.at[pl.ds(base + half, half)], ssb)
        sa.wait()
        sb.wait()

    return gk(idx_flat, table)


def _dot_t(x, w):
    return lax.dot_general(x, w, (((1,), (1,)), ((), ())),
                           preferred_element_type=jnp.float32)


def _tc_body(chars_ref, words_ref, ctab_ref, wcat_ref, bconv_ref,
             exp_ref, cmod_ref,
             wg0_ref, bg0_ref, wt0_ref, bt0_ref,
             wg1_ref, bg1_ref, wt1_ref, bt1_ref,
             out_ref, m_ref):
    @pl.when(pl.program_id(0) == 0)
    def _():
        m_ref[...] = jnp.zeros(((KW + 1) * NF, 2 * NF), jnp.bfloat16)
        mt = jnp.dot(ctab_ref[...], wcat_ref[...],
                     preferred_element_type=jnp.float32)
        for k in range(KW):
            blk = mt[:, k * NF:(k + 1) * NF].astype(jnp.bfloat16)
            m_ref[pl.ds(k * NF, VC), 0:NF] = blk
            m_ref[pl.ds((k + 1) * NF, VC), NF:2 * NF] = blk

    chars_bf = chars_ref[...].astype(jnp.bfloat16)
    chars_rep = jnp.dot(chars_bf, exp_ref[...],
                        preferred_element_type=jnp.float32
                        ).astype(jnp.bfloat16)
    oh2 = jnp.where(chars_rep == cmod_ref[...],
                    jnp.bfloat16(1), jnp.bfloat16(0))
    m = m_ref[...]

    acc = None
    for p in range(NPOS // 2):
        pr = lax.dot_general(oh2[:, 2 * p * NF:(2 * p + KW + 1) * NF], m,
                             (((1,), (0,)), ((), ())),
                             preferred_element_type=jnp.float32)
        acc = pr if acc is None else jnp.maximum(acc, pr)
    acc = jnp.maximum(acc[:, :NF], acc[:, NF:])
    ce = jnp.maximum(acc + bconv_ref[...], 0.0)

    x = jnp.concatenate([words_ref[...], ce], axis=1)
    for wg, bg, wt, bt in ((wg0_ref, bg0_ref, wt0_ref, bt0_ref),
                           (wg1_ref, bg1_ref, wt1_ref, bt1_ref)):
        xb = x.astype(jnp.bfloat16)
        g = jax.nn.sigmoid(_dot_t(xb, wg[...]) + bg[...])
        t = jnp.maximum(_dot_t(xb, wt[...]) + bt[...], 0.0)
        x = g * t + (1.0 - g) * x
    out_ref[...] = x


def _tc_fused(chars2, word_rows, ctab, wcat, bconv, expander, cmod,
              wg0, bg0, wt0, bt0, wg1, bg1, wt1, bt1):
    const = lambda i: (0, 0)
    row = lambda i: (i, 0)
    return pl.pallas_call(
        _tc_body,
        grid=(N // TT,),
        in_specs=[
            pl.BlockSpec((TT, L), row),
            pl.BlockSpec((TT, DW), row),
            pl.BlockSpec((VC, DC), const),
            pl.BlockSpec((DC, KW * NF), const),
            pl.BlockSpec((1, NF), const),
            pl.BlockSpec((L, L * NF), const),
            pl.BlockSpec((1, L * NF), const),
            pl.BlockSpec((D, D), const),
            pl.BlockSpec((1, D), const),
            pl.BlockSpec((D, D), const),
            pl.BlockSpec((1, D), const),
            pl.BlockSpec((D, D), const),
            pl.BlockSpec((1, D), const),
            pl.BlockSpec((D, D), const),
            pl.BlockSpec((1, D), const),
        ],
        out_specs=pl.BlockSpec((TT, D), row),
        out_shape=jax.ShapeDtypeStruct((N, D), jnp.float32),
        scratch_shapes=[pltpu.VMEM(((KW + 1) * NF, 2 * NF), jnp.bfloat16)],
    )(chars2, word_rows, ctab, wcat, bconv, expander, cmod,
      wg0, bg0, wt0, bt0, wg1, bg1, wt1, bt1)


def kernel(word_idxs, char_idxs, word_table, char_table, W_conv, b_conv,
           Wt0, bt0, Wg0, bg0, Wt1, bt1, Wg1, bg1):
    widx = word_idxs.reshape(N).astype(jnp.int32)
    word_rows = _word_gather(widx, word_table)

    chars2 = char_idxs.reshape(N, L).astype(jnp.int32)
    wcat = jnp.transpose(W_conv[:, :, 0, :], (1, 2, 0)).reshape(DC, KW * NF)
    bconv = b_conv.reshape(1, NF)
    cols = np.arange(L * NF)
    expander = jnp.asarray(
        (cols // NF == np.arange(L)[:, None]).astype(np.float32),
        dtype=jnp.bfloat16)
    cmod = jnp.asarray((cols % NF).astype(np.float32)[None, :],
                       dtype=jnp.bfloat16)
    emb = _tc_fused(
        chars2, word_rows, char_table, wcat, bconv, expander, cmod,
        Wg0.astype(jnp.bfloat16), bg0.reshape(1, D),
        Wt0.astype(jnp.bfloat16), bt0.reshape(1, D),
        Wg1.astype(jnp.bfloat16), bg1.reshape(1, D),
        Wt1.astype(jnp.bfloat16), bt1.reshape(1, D),
    )
    return emb.reshape(B, S, D)


# speedup vs baseline: 1.2447x; 1.0324x over previous
import functools

import numpy as np
import jax
import jax.numpy as jnp
from jax import lax
from jax.experimental import pallas as pl
from jax.experimental.pallas import tpu as pltpu
from jax.experimental.pallas import tpu_sc as plsc

B, S, L = 64, 400, 16
VW, DW = 100000, 128
VC, DC = 96, 64
NF = 128
KW = 5
D = DW + NF
N = B * S
NW = 32
RPW = N // NW
TT = 2048
NPOS = L - KW + 1


def _word_gather(idx_flat, table):
    mesh = plsc.VectorSubcoreMesh(core_axis_name="c", subcore_axis_name="s")

    half = RPW // 2

    @functools.partial(
        pl.kernel,
        out_type=jax.ShapeDtypeStruct((N, DW), jnp.float32),
        mesh=mesh,
        scratch_types=[
            pltpu.VMEM((RPW,), jnp.int32),
            pltpu.VMEM((half, DW), jnp.float32),
            pltpu.VMEM((half, DW), jnp.float32),
            pltpu.SemaphoreType.DMA,
            pltpu.SemaphoreType.DMA,
            pltpu.SemaphoreType.DMA,
            pltpu.SemaphoreType.DMA,
        ],
    )
    def gk(idx_hbm, table_hbm, out_hbm, idx_v, ra, rb, sga, sgb, ssa, ssb):
        wid = lax.axis_index("s") * 2 + lax.axis_index("c")
        base = wid * RPW
        pltpu.sync_copy(idx_hbm.at[pl.ds(base, RPW)], idx_v)
        ga = pltpu.async_copy(table_hbm.at[idx_v.at[pl.ds(0, half)]], ra, sga)
        gb = pltpu.async_copy(table_hbm.at[idx_v.at[pl.ds(half, half)]], rb, sgb)
        ga.wait()
        sa = pltpu.async_copy(ra, out_hbm.at[pl.ds(base, half)], ssa)
        gb.wait()
        sb = pltpu.async_copy(rb, out_hbm.at[pl.ds(base + half, half)], ssb)
        sa.wait()
        sb.wait()

    return gk(idx_flat, table)


def _dot_t(x, w):
    return lax.dot_general(x, w, (((1,), (1,)), ((), ())),
                           preferred_element_type=jnp.float32)


def _tc_body(chars_ref, words_ref, ctab_ref, wcat_ref, bconv_ref,
             exp_ref, cmod_ref,
             wg0_ref, bg0_ref, wt0_ref, bt0_ref,
             wg1_ref, bg1_ref, wt1_ref, bt1_ref,
             out_ref, m_ref):
    @pl.when(pl.program_id(0) == 0)
    def _():
        m_ref[...] = jnp.zeros(((KW + 1) * NF, 2 * NF), jnp.bfloat16)
        mt = jnp.dot(ctab_ref[...], wcat_ref[...],
                     preferred_element_type=jnp.float32)
        for k in range(KW):
            blk = mt[:, k * NF:(k + 1) * NF].astype(jnp.bfloat16)
            m_ref[pl.ds(k * NF, VC), 0:NF] = blk
            m_ref[pl.ds((k + 1) * NF, VC), NF:2 * NF] = blk

    chars_bf = chars_ref[...].astype(jnp.bfloat16)
    chars_rep = jnp.dot(chars_bf, exp_ref[...],
                        preferred_element_type=jnp.float32
                        ).astype(jnp.bfloat16)
    oh2 = jnp.where(chars_rep == cmod_ref[...],
                    jnp.bfloat16(1), jnp.bfloat16(0))
    m = m_ref[...]

    acc = None
    for p in range(NPOS // 2):
        pr = lax.dot_general(oh2[:, 2 * p * NF:(2 * p + KW + 1) * NF], m,
                             (((1,), (0,)), ((), ())),
                             preferred_element_type=jnp.float32)
        acc = pr if acc is None else jnp.maximum(acc, pr)
    acc = jnp.maximum(acc[:, :NF], acc[:, NF:])
    ce = jnp.maximum(acc + bconv_ref[...], 0.0)

    x = jnp.concatenate([words_ref[...], ce], axis=1)
    for wg, bg, wt, bt in ((wg0_ref, bg0_ref, wt0_ref, bt0_ref),
                           (wg1_ref, bg1_ref, wt1_ref, bt1_ref)):
        xb = x.astype(jnp.bfloat16)
        g = jax.nn.sigmoid(_dot_t(xb, wg[...]) + bg[...])
        t = jnp.maximum(_dot_t(xb, wt[...]) + bt[...], 0.0)
        x = g * t + (1.0 - g) * x
    out_ref[...] = x


def _tc_fused(chars2, word_rows, ctab, wcat, bconv, expander, cmod,
              wg0, bg0, wt0, bt0, wg1, bg1, wt1, bt1):
    const = lambda i: (0, 0)
    row = lambda i: (i, 0)
    return pl.pallas_call(
        _tc_body,
        grid=(N // TT,),
        in_specs=[
            pl.BlockSpec((TT, L), row),
            pl.BlockSpec((TT, DW), row),
            pl.BlockSpec((VC, DC), const),
            pl.BlockSpec((DC, KW * NF), const),
            pl.BlockSpec((1, NF), const),
            pl.BlockSpec((L, L * NF), const),
            pl.BlockSpec((1, L * NF), const),
            pl.BlockSpec((D, D), const),
            pl.BlockSpec((1, D), const),
            pl.BlockSpec((D, D), const),
            pl.BlockSpec((1, D), const),
            pl.BlockSpec((D, D), const),
            pl.BlockSpec((1, D), const),
            pl.BlockSpec((D, D), const),
            pl.BlockSpec((1, D), const),
        ],
        out_specs=pl.BlockSpec((TT, D), row),
        out_shape=jax.ShapeDtypeStruct((N, D), jnp.float32),
        scratch_shapes=[pltpu.VMEM(((KW + 1) * NF, 2 * NF), jnp.bfloat16)],
    )(chars2, word_rows, ctab, wcat, bconv, expander, cmod,
      wg0, bg0, wt0, bt0, wg1, bg1, wt1, bt1)


def kernel(word_idxs, char_idxs, word_table, char_table, W_conv, b_conv,
           Wt0, bt0, Wg0, bg0, Wt1, bt1, Wg1, bg1):
    widx = word_idxs.reshape(N).astype(jnp.int32)
    word_rows = _word_gather(widx, word_table)

    chars2 = char_idxs.reshape(N, L).astype(jnp.int32)
    wcat = jnp.transpose(W_conv[:, :, 0, :], (1, 2, 0)).reshape(DC, KW * NF)
    bconv = b_conv.reshape(1, NF)
    cols = np.arange(L * NF)
    expander = jnp.asarray(
        (cols // NF == np.arange(L)[:, None]).astype(np.float32),
        dtype=jnp.bfloat16)
    cmod = jnp.asarray((cols % NF).astype(np.float32)[None, :],
                       dtype=jnp.bfloat16)
    emb = _tc_fused(
        chars2, word_rows, char_table, wcat, bconv, expander, cmod,
        Wg0.astype(jnp.bfloat16), bg0.reshape(1, D),
        Wt0.astype(jnp.bfloat16), bt0.reshape(1, D),
        Wg1.astype(jnp.bfloat16), bg1.reshape(1, D),
        Wt1.astype(jnp.bfloat16), bt1.reshape(1, D),
    )
    return emb.reshape(B, S, D)
